# baseline (speedup 1.0000x reference)
import functools

import jax
import jax.numpy as jnp
from jax import lax
from jax.experimental import pallas as pl
from jax.experimental.pallas import tpu as pltpu
from jax.experimental.pallas import tpu_sc as plsc

_B = 16384
_D = 16
_NV = 1_000_000
_TILES = -(-_NV // 128)

_info = plsc.get_sparse_core_info()
_NC = _info.num_cores
_NS = _info.num_subcores
_L = _info.num_lanes
_NW = _NC * _NS
_PER_W = _B // _NW
_GROUPS = _PER_W // _L

_TPW = _TILES // _NW
_CHUNK_T = 16
_CHUNK_V = _CHUNK_T * 128
_NCHUNK = 16
_CAP = 1536

_mesh = plsc.VectorSubcoreMesh(core_axis_name="c", subcore_axis_name="s")
_params = pltpu.CompilerParams(needs_layout_passes=False)
_params_lin = pltpu.CompilerParams(
    needs_layout_passes=False, use_tc_tiling_on_sc=False)


def _make_extract_kernel():
    @functools.partial(
        pl.kernel,
        mesh=_mesh,
        out_type=(jax.ShapeDtypeStruct((_NW * _CAP * _D,), jnp.float32),
                  jax.ShapeDtypeStruct((_NW * _CAP,), jnp.int32),
                  jax.ShapeDtypeStruct((_NW * _CAP * _D,), jnp.float32),
                  jax.ShapeDtypeStruct((_NW * _CAP,), jnp.int32)),
        compiler_params=_params,
        scratch_types=[
            pltpu.VMEM((_B,), jnp.int32),
            pltpu.VMEM((2, _CHUNK_T, 8, 128), jnp.float32),
            pltpu.VMEM((2, _CHUNK_T, 8, 128), jnp.float32),
            pltpu.VMEM((_CAP + 4 * _L,), jnp.int32),
            pltpu.VMEM((_CAP + 4 * _L,), jnp.int32),
            pltpu.VMEM((_CAP + 4 * _L,), jnp.int32),
            pltpu.VMEM((_CAP + 4 * _L,), jnp.int32),
            pltpu.VMEM((_CAP * _D,), jnp.float32),
            pltpu.SemaphoreType.DMA,
            pltpu.SemaphoreType.DMA,
        ],
    )
    def extract_kernel(uidx_hbm, iidx_hbm, utab_hbm, itab_hbm,
                       ustage_hbm, upos_hbm, istage_hbm, ipos_hbm,
                       idx_v, cb0_v, cb1_v, mv_v, mp_v, vls_v, sls_v,
                       stage_v, sem0, sem1):
        wid = lax.axis_index("s") * _NC + lax.axis_index("c")
        start_t = wid * _TPW
        nt = jnp.where(wid == _NW - 1, _TILES - (_NW - 1) * _TPW, _TPW)
        end_t = start_t + nt
        lo = start_t * 128
        hi = end_t * 128
        lane = lax.iota(jnp.int32, _L)
        aconsts = [jnp.full((_L,), d // 8, jnp.int32) for d in range(_D)]
        cconsts = [jnp.full((_L,), d % 8, jnp.int32) for d in range(_D)]

        def phase(idx_hbm, tab_hbm, stage_hbm, pos_hbm):
            neg1 = jnp.full((_L,), -1, jnp.int32)

            def init_b(k, c):
                mp_v[pl.ds(k * _L, _L)] = neg1
                mv_v[pl.ds(k * _L, _L)] = neg1
                return c
            lax.fori_loop(0, (_CAP + 4 * _L) // _L, init_b, 0)

            pltpu.sync_copy(idx_hbm, idx_v)

            def scan_b(q, off):
                vs, ms, cnts = [], [], []
                for t in range(4):
                    v = idx_v[pl.ds((q * 4 + t) * _L, _L)]
                    m = (v >= lo) & (v < hi)
                    vs.append(v)
                    ms.append(m)
                    cnts.append(plsc.all_reduce_population_count(m)[0])
                o = off
                for t in range(4):
                    plsc.store_compressed(mv_v.at[pl.ds(o, _L)],
                                          vs[t], mask=ms[t])
                    plsc.store_compressed(mp_v.at[pl.ds(o, _L)],
                                          (q * 4 + t) * _L + lane,
                                          mask=ms[t])
                    o = o + cnts[t]
                return o

            off = lax.fori_loop(0, _B // (4 * _L), scan_b, 0)
            ng4 = (off + 4 * _L - 1) // (4 * _L)

            def chunk_lo(ct):
                t0 = jnp.minimum(start_t + ct * _CHUNK_T, end_t - _CHUNK_T)
                return t0 * 128

            def start_chunk(ct, cb, sem):
                base = chunk_lo(ct)
                for a in range(2):
                    for t in range(_CHUNK_T):
                        voff = pl.multiple_of(base + t * 128, 128)
                        pltpu.async_copy(
                            tab_hbm.at[pl.ds(a * 8, 8), pl.ds(voff, 128)],
                            cb.at[a, t], sem)

            def drain(cb, sem):
                for _ in range(2 * _CHUNK_T):
                    pltpu.make_async_copy(
                        tab_hbm.at[pl.ds(0, 8), pl.ds(0, 128)],
                        cb.at[0, 0], sem).wait()

            def process(cb, c_lo):
                def sub_b(q, cc):
                    vs4, ms4, cnts4 = [], [], []
                    for t in range(4):
                        vs = mv_v[pl.ds((q * 4 + t) * _L, _L)]
                        inm = (vs >= c_lo) & (vs < c_lo + _CHUNK_V)
                        vs4.append(vs - c_lo)
                        ms4.append(inm)
                        cnts4.append(
                            plsc.all_reduce_population_count(inm)[0])
                    o = cc
                    for t in range(4):
                        plsc.store_compressed(vls_v.at[pl.ds(o, _L)],
                                              vs4[t], mask=ms4[t])
                        plsc.store_compressed(sls_v.at[pl.ds(o, _L)],
                                              (q * 4 + t) * _L + lane,
                                              mask=ms4[t])
                        o = o + cnts4[t]
                    return o

                ccount = lax.fori_loop(0, ng4, sub_b, 0)
                ng2 = (ccount + _L - 1) // _L

                def ext_b(k, c2):
                    vl = vls_v[pl.ds(k * _L, _L)]
                    sl = sls_v[pl.ds(k * _L, _L)]
                    valid = (k * _L + lane) < ccount
                    vt = vl >> 7
                    ve = vl & 127
                    sbase = sl * _D
                    for d in range(_D):
                        vals = plsc.load_gather(
                            cb, [aconsts[d], vt, cconsts[d], ve],
                            mask=valid)
                        plsc.store_scatter(
                            stage_v, [sbase + d], vals, mask=valid)
                    return c2

                lax.fori_loop(0, ng2, ext_b, 0)

            start_chunk(0, cb0_v, sem0)

            def pair_b(h, c):
                ct0 = 2 * h
                start_chunk(ct0 + 1, cb1_v, sem1)
                drain(cb0_v, sem0)
                process(cb0_v, chunk_lo(ct0))
                start_chunk(ct0 + 2, cb0_v, sem0)
                drain(cb1_v, sem1)
                process(cb1_v, chunk_lo(ct0 + 1))
                return c

            lax.fori_loop(0, _NCHUNK // 2, pair_b, 0)
            drain(cb0_v, sem0)

            pltpu.sync_copy(
                stage_v, stage_hbm.at[pl.ds(wid * _CAP * _D, _CAP * _D)])
            pltpu.sync_copy(
                mp_v.at[pl.ds(0, _CAP)], pos_hbm.at[pl.ds(wid * _CAP, _CAP)])

        phase(uidx_hbm, utab_hbm, ustage_hbm, upos_hbm)
        phase(iidx_hbm, itab_hbm, istage_hbm, ipos_hbm)

    return extract_kernel


def _make_scatter_kernel():
    @functools.partial(
        pl.kernel,
        mesh=_mesh,
        out_type=(jax.ShapeDtypeStruct((_B, _D), jnp.float32),
                  jax.ShapeDtypeStruct((_B, _D), jnp.float32)),
        compiler_params=_params_lin,
        scratch_types=[
            pltpu.VMEM((_CAP, _D), jnp.float32),
            pltpu.VMEM((_CAP,), jnp.int32),
            pltpu.SemaphoreType.DMA,
        ],
    )
    def scatter_kernel(ustage_hbm, upos_hbm, istage_hbm, ipos_hbm,
                       urows_hbm, irows_hbm, st_v, pos_v, sem):
        wid = lax.axis_index("s") * _NC + lax.axis_index("c")
        for stage_hbm, pos_hbm, rows_hbm in (
                (ustage_hbm, upos_hbm, urows_hbm),
                (istage_hbm, ipos_hbm, irows_hbm)):
            pltpu.sync_copy(stage_hbm.at[pl.ds(wid * _CAP, _CAP), :], st_v)
            pltpu.sync_copy(pos_hbm.at[pl.ds(wid * _CAP, _CAP)], pos_v)
            pltpu.async_copy(
                st_v,
                rows_hbm.at[plsc.Indices(pos_v, ignored_value=-1)],
                sem).wait()

    return scatter_kernel


def _make_combine_kernel():
    @functools.partial(
        pl.kernel,
        mesh=_mesh,
        out_type=jax.ShapeDtypeStruct((_B,), jnp.float32),
        compiler_params=_params_lin,
        scratch_types=[
            pltpu.VMEM((_PER_W, _D), jnp.float32),
            pltpu.VMEM((_PER_W, _D), jnp.float32),
            pltpu.VMEM((2 * _L,), jnp.float32),
            pltpu.VMEM((_PER_W,), jnp.float32),
        ],
    )
    def combine_kernel(urows_hbm, irows_hbm, wb_hbm, out_hbm,
                       u_v, i_v, wb_v, out_v):
        wid = lax.axis_index("s") * _NC + lax.axis_index("c")
        base = wid * _PER_W
        pltpu.sync_copy(urows_hbm.at[pl.ds(base, _PER_W), :], u_v)
        pltpu.sync_copy(irows_hbm.at[pl.ds(base, _PER_W), :], i_v)
        pltpu.sync_copy(wb_hbm, wb_v)
        wvec = wb_v[pl.ds(0, _L)]
        bvec = wb_v[pl.ds(_L, _L)]
        lane = lax.iota(jnp.int32, _L)

        def group_b(g, c):
            rows = g * _L + lane
            acc = bvec
            for d in range(_D):
                dvec = jnp.full((_L,), d, jnp.int32)
                uv = plsc.load_gather(u_v, [rows, dvec])
                iv = plsc.load_gather(i_v, [rows, dvec])
                acc = acc + (uv * iv) * wvec[d]
            out_v[pl.ds(g * _L, _L)] = 1.0 / (1.0 + jnp.exp(-acc))
            return c

        lax.fori_loop(0, _GROUPS, group_b, 0)
        pltpu.sync_copy(out_v, out_hbm.at[pl.ds(base, _PER_W)])

    return combine_kernel


_extract = _make_extract_kernel()
_scatter = _make_scatter_kernel()
_combine = _make_combine_kernel()


def kernel(user_input, item_input, user_table, item_table, W, b):
    uidx = user_input.reshape(_B).astype(jnp.int32)
    iidx = item_input.reshape(_B).astype(jnp.int32)
    wb = jnp.concatenate(
        [W.reshape(_D), jnp.broadcast_to(b.astype(jnp.float32), (_L,))])
    ustage, upos, istage, ipos = _extract(
        uidx, iidx, user_table.T, item_table.T)
    urows, irows = _scatter(
        ustage.reshape(_NW * _CAP, _D), upos,
        istage.reshape(_NW * _CAP, _D), ipos)
    out = _combine(urows, irows, wb)
    return out.reshape(_B, 1)


# speedup vs baseline: 1.0185x; 1.0185x over previous
import functools

import jax
import jax.numpy as jnp
from jax import lax
from jax.experimental import pallas as pl
from jax.experimental.pallas import tpu as pltpu
from jax.experimental.pallas import tpu_sc as plsc

_B = 16384
_D = 16
_NV = 1_000_000
_TILES = -(-_NV // 128)

_info = plsc.get_sparse_core_info()
_NC = _info.num_cores
_NS = _info.num_subcores
_L = _info.num_lanes
_NW = _NC * _NS
_PER_W = _B // _NW
_GROUPS = _PER_W // _L

_TPW = _TILES // _NW
_CHUNK_T = 16
_CHUNK_V = _CHUNK_T * 128
_NCHUNK = 16
_CAP = 1536

_mesh = plsc.VectorSubcoreMesh(core_axis_name="c", subcore_axis_name="s")
_params = pltpu.CompilerParams(needs_layout_passes=False)
_params_lin = pltpu.CompilerParams(
    needs_layout_passes=False, use_tc_tiling_on_sc=False)


def _make_extract_kernel():
    @functools.partial(
        pl.kernel,
        mesh=_mesh,
        out_type=(jax.ShapeDtypeStruct((_NW * _CAP * _D,), jnp.float32),
                  jax.ShapeDtypeStruct((_NW * _CAP,), jnp.int32),
                  jax.ShapeDtypeStruct((_NW * _CAP * _D,), jnp.float32),
                  jax.ShapeDtypeStruct((_NW * _CAP,), jnp.int32)),
        compiler_params=_params,
        scratch_types=[
            pltpu.VMEM((_B,), jnp.int32),
            pltpu.VMEM((_D, _CHUNK_V), jnp.float32),
            pltpu.VMEM((_D, _CHUNK_V), jnp.float32),
            pltpu.VMEM((_CAP + 4 * _L,), jnp.int32),
            pltpu.VMEM((_CAP + 4 * _L,), jnp.int32),
            pltpu.VMEM((_CAP + 4 * _L,), jnp.int32),
            pltpu.VMEM((_CAP + 4 * _L,), jnp.int32),
            pltpu.VMEM((_CAP * _D,), jnp.float32),
            pltpu.SemaphoreType.DMA,
            pltpu.SemaphoreType.DMA,
        ],
    )
    def extract_kernel(uidx_hbm, iidx_hbm, utab_hbm, itab_hbm,
                       ustage_hbm, upos_hbm, istage_hbm, ipos_hbm,
                       idx_v, cb0_v, cb1_v, mv_v, mp_v, vls_v, sls_v,
                       stage_v, sem0, sem1):
        wid = lax.axis_index("s") * _NC + lax.axis_index("c")
        start_t = wid * _TPW
        nt = jnp.where(wid == _NW - 1, _TILES - (_NW - 1) * _TPW, _TPW)
        end_t = start_t + nt
        lo = start_t * 128
        hi = end_t * 128
        lane = lax.iota(jnp.int32, _L)
        dconsts = [jnp.full((_L,), d, jnp.int32) for d in range(_D)]

        def phase(idx_hbm, tab_hbm, stage_hbm, pos_hbm):
            neg1 = jnp.full((_L,), -1, jnp.int32)

            def init_b(k, c):
                mp_v[pl.ds(k * _L, _L)] = neg1
                mv_v[pl.ds(k * _L, _L)] = neg1
                return c
            lax.fori_loop(0, (_CAP + 4 * _L) // _L, init_b, 0)

            pltpu.sync_copy(idx_hbm, idx_v)

            def scan_b(q, off):
                vs, ms, cnts = [], [], []
                for t in range(4):
                    v = idx_v[pl.ds((q * 4 + t) * _L, _L)]
                    m = (v >= lo) & (v < hi)
                    vs.append(v)
                    ms.append(m)
                    cnts.append(plsc.all_reduce_population_count(m)[0])
                o = off
                for t in range(4):
                    plsc.store_compressed(mv_v.at[pl.ds(o, _L)],
                                          vs[t], mask=ms[t])
                    plsc.store_compressed(mp_v.at[pl.ds(o, _L)],
                                          (q * 4 + t) * _L + lane,
                                          mask=ms[t])
                    o = o + cnts[t]
                return o

            off = lax.fori_loop(0, _B // (4 * _L), scan_b, 0)
            ng4 = (off + 4 * _L - 1) // (4 * _L)

            def chunk_lo(ct):
                t0 = jnp.minimum(start_t + ct * _CHUNK_T, end_t - _CHUNK_T)
                return t0 * 128

            def start_chunk(ct, cb, sem):
                voff = pl.multiple_of(chunk_lo(ct), 128)
                pltpu.async_copy(
                    tab_hbm.at[pl.ds(0, 8), pl.ds(voff, _CHUNK_V)],
                    cb.at[pl.ds(0, 8), :], sem)
                pltpu.async_copy(
                    tab_hbm.at[pl.ds(8, 8), pl.ds(voff, _CHUNK_V)],
                    cb.at[pl.ds(8, 8), :], sem)

            def drain(cb, sem):
                pltpu.make_async_copy(
                    tab_hbm.at[:, pl.ds(0, _CHUNK_V)], cb, sem).wait()

            def process(cb, c_lo):
                def sub_b(q, cc):
                    vs4, ms4, cnts4 = [], [], []
                    for t in range(4):
                        vs = mv_v[pl.ds((q * 4 + t) * _L, _L)]
                        inm = (vs >= c_lo) & (vs < c_lo + _CHUNK_V)
                        vs4.append(vs - c_lo)
                        ms4.append(inm)
                        cnts4.append(
                            plsc.all_reduce_population_count(inm)[0])
                    o = cc
                    for t in range(4):
                        plsc.store_compressed(vls_v.at[pl.ds(o, _L)],
                                              vs4[t], mask=ms4[t])
                        plsc.store_compressed(sls_v.at[pl.ds(o, _L)],
                                              (q * 4 + t) * _L + lane,
                                              mask=ms4[t])
                        o = o + cnts4[t]
                    return o

                ccount = lax.fori_loop(0, ng4, sub_b, 0)
                ng2 = (ccount + _L - 1) // _L

                def ext_b(k, c2):
                    vl = vls_v[pl.ds(k * _L, _L)]
                    sl = sls_v[pl.ds(k * _L, _L)]
                    valid = (k * _L + lane) < ccount
                    sbase = sl * _D
                    for d in range(_D):
                        vals = plsc.load_gather(
                            cb, [dconsts[d], vl], mask=valid)
                        plsc.store_scatter(
                            stage_v, [sbase + d], vals, mask=valid)
                    return c2

                lax.fori_loop(0, ng2, ext_b, 0)

            start_chunk(0, cb0_v, sem0)

            def pair_b(h, c):
                ct0 = 2 * h
                start_chunk(ct0 + 1, cb1_v, sem1)
                drain(cb0_v, sem0)
                process(cb0_v, chunk_lo(ct0))
                start_chunk(ct0 + 2, cb0_v, sem0)
                drain(cb1_v, sem1)
                process(cb1_v, chunk_lo(ct0 + 1))
                return c

            lax.fori_loop(0, _NCHUNK // 2, pair_b, 0)
            drain(cb0_v, sem0)

            pltpu.sync_copy(
                stage_v, stage_hbm.at[pl.ds(wid * _CAP * _D, _CAP * _D)])
            pltpu.sync_copy(
                mp_v.at[pl.ds(0, _CAP)], pos_hbm.at[pl.ds(wid * _CAP, _CAP)])

        phase(uidx_hbm, utab_hbm, ustage_hbm, upos_hbm)
        phase(iidx_hbm, itab_hbm, istage_hbm, ipos_hbm)

    return extract_kernel


def _make_scatter_kernel():
    @functools.partial(
        pl.kernel,
        mesh=_mesh,
        out_type=(jax.ShapeDtypeStruct((_B, _D), jnp.float32),
                  jax.ShapeDtypeStruct((_B, _D), jnp.float32)),
        compiler_params=_params_lin,
        scratch_types=[
            pltpu.VMEM((_CAP, _D), jnp.float32),
            pltpu.VMEM((_CAP,), jnp.int32),
            pltpu.SemaphoreType.DMA,
        ],
    )
    def scatter_kernel(ustage_hbm, upos_hbm, istage_hbm, ipos_hbm,
                       urows_hbm, irows_hbm, st_v, pos_v, sem):
        wid = lax.axis_index("s") * _NC + lax.axis_index("c")
        for stage_hbm, pos_hbm, rows_hbm in (
                (ustage_hbm, upos_hbm, urows_hbm),
                (istage_hbm, ipos_hbm, irows_hbm)):
            pltpu.sync_copy(stage_hbm.at[pl.ds(wid * _CAP, _CAP), :], st_v)
            pltpu.sync_copy(pos_hbm.at[pl.ds(wid * _CAP, _CAP)], pos_v)
            pltpu.async_copy(
                st_v,
                rows_hbm.at[plsc.Indices(pos_v, ignored_value=-1)],
                sem).wait()

    return scatter_kernel


def _make_combine_kernel():
    @functools.partial(
        pl.kernel,
        mesh=_mesh,
        out_type=jax.ShapeDtypeStruct((_B,), jnp.float32),
        compiler_params=_params_lin,
        scratch_types=[
            pltpu.VMEM((_PER_W, _D), jnp.float32),
            pltpu.VMEM((_PER_W, _D), jnp.float32),
            pltpu.VMEM((2 * _L,), jnp.float32),
            pltpu.VMEM((_PER_W,), jnp.float32),
        ],
    )
    def combine_kernel(urows_hbm, irows_hbm, wb_hbm, out_hbm,
                       u_v, i_v, wb_v, out_v):
        wid = lax.axis_index("s") * _NC + lax.axis_index("c")
        base = wid * _PER_W
        pltpu.sync_copy(urows_hbm.at[pl.ds(base, _PER_W), :], u_v)
        pltpu.sync_copy(irows_hbm.at[pl.ds(base, _PER_W), :], i_v)
        pltpu.sync_copy(wb_hbm, wb_v)
        wvec = wb_v[pl.ds(0, _L)]
        bvec = wb_v[pl.ds(_L, _L)]
        lane = lax.iota(jnp.int32, _L)

        def group_b(g, c):
            rows = g * _L + lane
            acc = bvec
            for d in range(_D):
                dvec = jnp.full((_L,), d, jnp.int32)
                uv = plsc.load_gather(u_v, [rows, dvec])
                iv = plsc.load_gather(i_v, [rows, dvec])
                acc = acc + (uv * iv) * wvec[d]
            out_v[pl.ds(g * _L, _L)] = 1.0 / (1.0 + jnp.exp(-acc))
            return c

        lax.fori_loop(0, _GROUPS, group_b, 0)
        pltpu.sync_copy(out_v, out_hbm.at[pl.ds(base, _PER_W)])

    return combine_kernel


_extract = _make_extract_kernel()
_scatter = _make_scatter_kernel()
_combine = _make_combine_kernel()


def kernel(user_input, item_input, user_table, item_table, W, b):
    uidx = user_input.reshape(_B).astype(jnp.int32)
    iidx = item_input.reshape(_B).astype(jnp.int32)
    wb = jnp.concatenate(
        [W.reshape(_D), jnp.broadcast_to(b.astype(jnp.float32), (_L,))])
    ustage, upos, istage, ipos = _extract(
        uidx, iidx, user_table.T, item_table.T)
    urows, irows = _scatter(
        ustage.reshape(_NW * _CAP, _D), upos,
        istage.reshape(_NW * _CAP, _D), ipos)
    out = _combine(urows, irows, wb)
    return out.reshape(_B, 1)
